# baseline (speedup 1.0000x reference)
import functools

import jax
import jax.numpy as jnp
from jax import lax
from jax.experimental import pallas as pl
from jax.experimental.pallas import tpu as pltpu
from jax.experimental.pallas import tpu_sc as plsc

CLASSES = 100000
DIMS = 64
BATCH = 16384
EPS = 1e-09

NC, NS, L = 2, 16, 16
NW = NC * NS
BPW = BATCH // NW
IDX_CHUNKS = BPW // 128


def _loss_body(pred_hbm, yt_hbm, pol_hbm, out_hbm,
               idx_v, hidx_v, rows_v, pred_v, ctab_v, stage_v, gsem, psem):
    wid = lax.axis_index("s") * NC + lax.axis_index("c")
    lane = lax.iota(jnp.int32, L)

    pltpu.sync_copy(yt_hbm.at[pl.ds(wid * IDX_CHUNKS, IDX_CHUNKS)], idx_v)
    for j in range(IDX_CHUNKS):
        for k in range(128 // L):
            hidx_v[j, pl.ds(k * L, L)] = idx_v[j, pl.ds(k * L, L)] >> 1

    pred_cp = pltpu.async_copy(
        pred_hbm.at[pl.ds(wid * (BPW // 2), BPW // 2)], pred_v, psem)
    gathers = [
        pltpu.async_copy(pol_hbm.at[hidx_v.at[j]],
                         rows_v.at[pl.ds(j * 128, 128)], gsem)
        for j in range(IDX_CHUNKS)
    ]
    for t in range(DIMS):
        ctab_v[t, :] = (lane + t) & (DIMS - 1)
    for g in gathers:
        g.wait()
    pred_cp.wait()

    half = jnp.float32(0.5)
    three_half = jnp.float32(1.5)
    one = jnp.float32(1.0)

    def group_body(g, acc):
        cvec = idx_v[g >> 3, pl.ds((g & 7) * L, L)]
        s = lane + g * L
        toff = (cvec & 1) << 6
        trow = s
        prow = s >> 1
        pcol0 = (s & 1) << 6
        dot = [None] * 4
        n1 = [None] * 4
        n2 = [None] * 4
        for t in range(DIMS):
            ct = ctab_v[t, :]
            pv = plsc.load_gather(pred_v, [prow, pcol0 + ct])
            tv = plsc.load_gather(rows_v, [trow, toff + ct])
            k = t & 3
            if dot[k] is None:
                dot[k], n1[k], n2[k] = pv * tv, pv * pv, tv * tv
            else:
                dot[k] = dot[k] + pv * tv
                n1[k] = n1[k] + pv * pv
                n2[k] = n2[k] + tv * tv
        dotv = (dot[0] + dot[1]) + (dot[2] + dot[3])
        n1v = (n1[0] + n1[1]) + (n1[2] + n1[3])
        n2v = (n2[0] + n2[1]) + (n2[2] + n2[3])
        prod = jnp.maximum(n1v * n2v, jnp.float32(1e-30))
        bits = plsc.bitcast(prod, jnp.int32)
        y = plsc.bitcast(jnp.int32(0x5F3759DF) - (bits >> 1), jnp.float32)
        for _ in range(3):
            y = y * (three_half - half * prod * y * y)
        scale = jnp.where(prod >= jnp.float32(EPS * EPS), y,
                          jnp.float32(1.0 / EPS))
        cos = dotv * scale
        e = one - cos
        return acc + e * e

    acc = lax.fori_loop(0, BPW // L, group_body,
                        jnp.zeros((L,), jnp.float32))
    stage_v[...] = acc * jnp.float32(1.0 / BATCH)
    pltpu.sync_copy(stage_v, out_hbm.at[wid])


_sc_loss = functools.partial(
    pl.kernel,
    mesh=plsc.VectorSubcoreMesh(core_axis_name="c", subcore_axis_name="s"),
    out_type=jax.ShapeDtypeStruct((NW, L), jnp.float32),
    compiler_params=pltpu.CompilerParams(
        needs_layout_passes=False, use_tc_tiling_on_sc=False),
    scratch_types=[
        pltpu.VMEM((IDX_CHUNKS, 128), jnp.int32),
        pltpu.VMEM((IDX_CHUNKS, 128), jnp.int32),
        pltpu.VMEM((BPW, 128), jnp.float32),
        pltpu.VMEM((BPW // 2, 128), jnp.float32),
        pltpu.VMEM((DIMS, L), jnp.int32),
        pltpu.VMEM((L,), jnp.float32),
        pltpu.SemaphoreType.DMA,
        pltpu.SemaphoreType.DMA,
    ],
)(_loss_body)


def kernel(y_pred, y_true, polars):
    yt = y_true.astype(jnp.int32).reshape(BATCH // 128, 128)
    pred2 = y_pred.reshape(BATCH // 2, 2 * DIMS)
    pol2 = polars.reshape(CLASSES // 2, 2 * DIMS)
    partials = _sc_loss(pred2, yt, pol2)
    return jnp.sum(partials)


# speedup vs baseline: 1.6259x; 1.6259x over previous
import functools

import jax
import jax.numpy as jnp
from jax import lax
from jax.experimental import pallas as pl
from jax.experimental.pallas import tpu as pltpu
from jax.experimental.pallas import tpu_sc as plsc

CLASSES = 100000
DIMS = 64
BATCH = 16384
EPS = 1e-09

NC, NS, L = 2, 16, 16
NW = NC * NS
BPW = BATCH // NW
HPW = BPW // 2


def _loss_body(pred_hbm, yt_hbm, pol_hbm, out_hbm,
               rows_v, pred_v, stage_v, rsem, psem):
    wid = lax.axis_index("s") * NC + lax.axis_index("c")
    base = wid * BPW
    lane = lax.iota(jnp.int32, L)

    pltpu.sync_copy(yt_hbm.at[pl.ds(wid * 8, 8)], pred_v.at[pl.ds(0, 8)])

    def rows_fire(g, c):
        civ = plsc.bitcast(pred_v[g >> 2, pl.ds((g & 3) * L, L)], jnp.int32)
        s0 = g * L
        for l in range(L):
            pltpu.make_async_copy(pol_hbm.at[civ[l]], rows_v.at[s0 + l],
                                  rsem).start()
        return c

    def pred_fire(i, c):
        pltpu.make_async_copy(pred_hbm.at[base + i],
                              pred_v.at[i & (HPW - 1)], psem).start()
        return c

    def rows_drain(i, c):
        pltpu.make_async_copy(pol_hbm.at[0], rows_v.at[i], rsem).wait()
        return c

    def pred_drain(i, c):
        pltpu.make_async_copy(pred_hbm.at[0], pred_v.at[0], psem).wait()
        return c

    lax.fori_loop(0, BPW // L, rows_fire, jnp.int32(0))
    lax.fori_loop(0, HPW, pred_fire, jnp.int32(0))
    lax.fori_loop(0, BPW, rows_drain, jnp.int32(0))
    lax.fori_loop(0, HPW, pred_drain, jnp.int32(0))

    half = jnp.float32(0.5)
    three_half = jnp.float32(1.5)
    one = jnp.float32(1.0)

    def make_group_body(pred_base):
        def group_body(g, acc):
            s = lane + g * L
            sp = s - pred_base
            dot = [None] * 4
            n1 = [None] * 4
            n2 = [None] * 4
            for t in range(DIMS):
                ct = (lane + t) & (DIMS - 1)
                pv = plsc.load_gather(pred_v, [sp, ct])
                tv = plsc.load_gather(rows_v, [s, ct])
                k = t & 3
                if dot[k] is None:
                    dot[k], n1[k], n2[k] = pv * tv, pv * pv, tv * tv
                else:
                    dot[k] = dot[k] + pv * tv
                    n1[k] = n1[k] + pv * pv
                    n2[k] = n2[k] + tv * tv
            dotv = (dot[0] + dot[1]) + (dot[2] + dot[3])
            n1v = (n1[0] + n1[1]) + (n1[2] + n1[3])
            n2v = (n2[0] + n2[1]) + (n2[2] + n2[3])
            prod = jnp.maximum(n1v * n2v, jnp.float32(1e-30))
            bits = plsc.bitcast(prod, jnp.int32)
            y = plsc.bitcast(jnp.int32(0x5F3759DF) - (bits >> 1),
                             jnp.float32)
            for _ in range(3):
                y = y * (three_half - half * prod * y * y)
            scale = jnp.where(prod >= jnp.float32(EPS * EPS), y,
                              jnp.float32(1.0 / EPS))
            cos = dotv * scale
            e = one - cos
            return acc + e * e
        return group_body

    acc = lax.fori_loop(0, HPW // L, make_group_body(0),
                        jnp.zeros((L,), jnp.float32))
    lax.fori_loop(HPW, BPW, pred_fire, jnp.int32(0))
    lax.fori_loop(0, HPW, pred_drain, jnp.int32(0))
    acc = lax.fori_loop(HPW // L, BPW // L, make_group_body(HPW), acc)

    stage_v[...] = acc * jnp.float32(1.0 / BATCH)
    pltpu.sync_copy(stage_v, out_hbm.at[wid])


_sc_loss = functools.partial(
    pl.kernel,
    mesh=plsc.VectorSubcoreMesh(core_axis_name="c", subcore_axis_name="s"),
    out_type=jax.ShapeDtypeStruct((NW, L), jnp.float32),
    compiler_params=pltpu.CompilerParams(needs_layout_passes=False),
    scratch_types=[
        pltpu.VMEM((BPW, DIMS), jnp.float32),
        pltpu.VMEM((HPW, DIMS), jnp.float32),
        pltpu.VMEM((L,), jnp.float32),
        pltpu.SemaphoreType.DMA,
        pltpu.SemaphoreType.DMA,
    ],
)(_loss_body)


def kernel(y_pred, y_true, polars):
    yt = lax.bitcast_convert_type(y_true.astype(jnp.int32),
                                  jnp.float32).reshape(BATCH // DIMS, DIMS)
    partials = _sc_loss(y_pred, yt, polars)
    return jnp.sum(partials)
